# Initial kernel scaffold; baseline (speedup 1.0000x reference)
import jax
import jax.numpy as jnp
from jax.experimental import pallas as pl


def kernel(x, edge_index, edge_attr, W1, b1, W2, b2):
    raise NotImplementedError("write your pallas kernel here")


# speedup vs baseline: 10.3372x; 10.3372x over previous
import functools

import jax
import jax.numpy as jnp
from jax import lax
from jax.experimental import pallas as pl
from jax.experimental.pallas import tpu as pltpu
from jax.experimental.pallas import tpu_sc as plsc

N = 10000
E = 320000
F_IN = 128
HID = 16
C = 40
CP = 48
NPAD = 10240

NC = 2
NS = 16
NW = NC * NS
EPW = E // NW
CHUNK = 80
NCHUNKS = EPW // CHUNK


def _mesh():
    return plsc.VectorSubcoreMesh(core_axis_name="c", subcore_axis_name="s")


def _bcast_lane(v16, t):
    idx = jnp.full((16, 1), t, jnp.int32)
    dnums = lax.GatherDimensionNumbers(
        offset_dims=(), collapsed_slice_dims=(0,), start_index_map=(0,))
    return lax.gather(v16, idx, dimension_numbers=dnums, slice_sizes=(1,),
                      mode=lax.GatherScatterMode.PROMISE_IN_BOUNDS)


@functools.partial(
    pl.kernel,
    out_type=jax.ShapeDtypeStruct((NC, NPAD), jnp.float32),
    mesh=_mesh(),
    scratch_types=[
        pltpu.VMEM((CHUNK,), jnp.int32),
        pltpu.VMEM((CHUNK,), jnp.float32),
        pltpu.VMEM((NPAD // NS,), jnp.float32),
        pltpu.VMEM_SHARED((NPAD,), jnp.float32),
    ],
)
def _sc_deg(col_hbm, ew_hbm, out_hbm, colc_v, ewc_v, zb_v, deg_s):
    c = lax.axis_index("c")
    s = lax.axis_index("s")
    wid = s * NC + c
    span = NPAD // NS

    def zero_body(i, _):
        zb_v[pl.ds(i * 16, 16)] = jnp.zeros((16,), jnp.float32)
        return 0

    lax.fori_loop(0, span // 16, zero_body, 0)
    pltpu.sync_copy(zb_v, deg_s.at[pl.ds(s * span, span)])
    plsc.subcore_barrier()

    base = wid * EPW

    def body(j, _):
        eb = base + j * CHUNK
        pltpu.sync_copy(col_hbm.at[pl.ds(eb, CHUNK)], colc_v)
        pltpu.sync_copy(ew_hbm.at[pl.ds(eb, CHUNK)], ewc_v)
        pltpu.sync_copy(ewc_v, deg_s.at[colc_v], add=True)
        return 0

    lax.fori_loop(0, NCHUNKS, body, 0)
    plsc.subcore_barrier()
    pltpu.sync_copy(deg_s.at[pl.ds(s * span, span)],
                    out_hbm.at[c, pl.ds(s * span, span)])


def _make_prop(D):
    rows_per_tile = NPAD // NS

    @functools.partial(
        pl.kernel,
        out_type=jax.ShapeDtypeStruct((NC, NPAD, D), jnp.float32),
        mesh=_mesh(),
        scratch_types=[
            pltpu.VMEM((CHUNK,), jnp.int32),
            pltpu.VMEM((CHUNK,), jnp.int32),
            pltpu.VMEM((CHUNK,), jnp.float32),
            pltpu.VMEM((CHUNK, D), jnp.float32),
            pltpu.VMEM((rows_per_tile, D), jnp.float32),
            pltpu.VMEM_SHARED((NPAD, D), jnp.float32),
            pltpu.SemaphoreType.DMA,
        ],
        compiler_params=pltpu.CompilerParams(use_tc_tiling_on_sc=False),
    )
    def prop(row_hbm, col_hbm, ew_hbm, tab_hbm, out_hbm,
             rowc_v, colc_v, ewc_v, rows_v, zb_v, agg_s, sem):
        c = lax.axis_index("c")
        s = lax.axis_index("s")
        wid = s * NC + c

        def zero_body(i, _):
            for u in range(D // 16):
                zb_v[i, pl.ds(u * 16, 16)] = jnp.zeros((16,), jnp.float32)
            return 0

        lax.fori_loop(0, rows_per_tile, zero_body, 0)
        r0 = s * rows_per_tile
        pltpu.sync_copy(zb_v, agg_s.at[pl.ds(r0, rows_per_tile), :])
        plsc.subcore_barrier()

        base = wid * EPW

        def body(j, _):
            eb = base + j * CHUNK
            pltpu.sync_copy(row_hbm.at[pl.ds(eb, CHUNK)], rowc_v)
            pltpu.sync_copy(col_hbm.at[pl.ds(eb, CHUNK)], colc_v)
            pltpu.sync_copy(ew_hbm.at[pl.ds(eb, CHUNK)], ewc_v)
            pltpu.async_copy(tab_hbm.at[rowc_v], rows_v, sem).wait()
            for g in range(CHUNK // 16):
                w16 = ewc_v[pl.ds(g * 16, 16)]
                for t in range(16):
                    k = g * 16 + t
                    wb = _bcast_lane(w16, t)
                    for u in range(D // 16):
                        sl = pl.ds(u * 16, 16)
                        rows_v[k, sl] = rows_v[k, sl] * wb
            pltpu.sync_copy(rows_v, agg_s.at[colc_v], add=True)
            return 0

        lax.fori_loop(0, NCHUNKS, body, 0)
        plsc.subcore_barrier()
        pltpu.sync_copy(agg_s.at[pl.ds(r0, rows_per_tile), :],
                        out_hbm.at[c, pl.ds(r0, rows_per_tile), :])

    return prop


_prop16 = _make_prop(HID)
_prop48 = _make_prop(CP)


def _tc1_body(x_ref, w1_ref, dp_ref, xws_ref, dis_ref):
    deg = dp_ref[0] + dp_ref[1]
    dis = jnp.where(deg > 0, lax.rsqrt(deg), 0.0)
    dis_ref[...] = dis
    xw = jnp.dot(x_ref[...], w1_ref[...], preferred_element_type=jnp.float32)
    xws_ref[...] = xw * dis


def _tc2_body(p_ref, dis_ref, b1_ref, w2_ref, hws_ref):
    dis = dis_ref[...]
    agg = (p_ref[0] + p_ref[1]) * dis
    h = jnp.maximum(agg + b1_ref[...], 0.0)
    hw = jnp.dot(h, w2_ref[...], preferred_element_type=jnp.float32)
    hws_ref[...] = hw * dis


def _tc3_body(p_ref, dis_ref, b2_ref, out_ref):
    z = (p_ref[0] + p_ref[1]) * dis_ref[...]
    z = z[:, :C] + b2_ref[...]
    m = jnp.max(z, axis=1, keepdims=True)
    lse = jnp.log(jnp.sum(jnp.exp(z - m), axis=1, keepdims=True)) + m
    out_ref[...] = z - lse


_tc1 = pl.pallas_call(
    _tc1_body,
    out_shape=(jax.ShapeDtypeStruct((N, HID), jnp.float32),
               jax.ShapeDtypeStruct((N, 1), jnp.float32)),
)
_tc2 = pl.pallas_call(
    _tc2_body,
    out_shape=jax.ShapeDtypeStruct((N, CP), jnp.float32),
)
_tc3 = pl.pallas_call(
    _tc3_body,
    out_shape=jax.ShapeDtypeStruct((N, C), jnp.float32),
)


def kernel(x, edge_index, edge_attr, W1, b1, W2, b2):
    row = edge_index[0]
    col = edge_index[1]
    deg_p = _sc_deg(col, edge_attr)
    xws, dis = _tc1(x, W1, deg_p[:, :N, None])
    p1 = _prop16(row, col, edge_attr, xws)[:, :N]
    w2p = jnp.zeros((HID, CP), jnp.float32).at[:, :C].set(W2)
    hws = _tc2(p1, dis, b1[None, :], w2p)
    p2 = _prop48(row, col, edge_attr, hws)[:, :N]
    return _tc3(p2, dis, b2[None, :])


# speedup vs baseline: 23.6305x; 2.2860x over previous
import functools

import jax
import jax.numpy as jnp
from jax import lax
from jax.experimental import pallas as pl
from jax.experimental.pallas import tpu as pltpu
from jax.experimental.pallas import tpu_sc as plsc

N = 10000
E = 320000
F_IN = 128
HID = 16
C = 40
CP = 48
NPAD = 10240

NC = 2
NS = 16
NW = NC * NS
CHUNK = 80
NCH = 128
EPAD = NW * NCH * CHUNK
RING = 4


def _mesh():
    return plsc.VectorSubcoreMesh(core_axis_name="c", subcore_axis_name="s")


def _bcast_lane(v16, t):
    idx = jnp.full((16, 1), t, jnp.int32)
    dnums = lax.GatherDimensionNumbers(
        offset_dims=(), collapsed_slice_dims=(0,), start_index_map=(0,))
    return lax.gather(v16, idx, dimension_numbers=dnums, slice_sizes=(1,),
                      mode=lax.GatherScatterMode.PROMISE_IN_BOUNDS)


@functools.partial(
    pl.kernel,
    out_type=jax.ShapeDtypeStruct((NC, NPAD), jnp.float32),
    mesh=_mesh(),
    scratch_types=[
        pltpu.VMEM((NCH, CHUNK), jnp.int32),
        pltpu.VMEM((NCH, CHUNK), jnp.float32),
        pltpu.VMEM((NPAD // NS,), jnp.float32),
        pltpu.VMEM_SHARED((NPAD,), jnp.float32),
        pltpu.SemaphoreType.DMA,
    ],
    compiler_params=pltpu.CompilerParams(use_tc_tiling_on_sc=False),
)
def _sc_deg(col_hbm, ew_hbm, out_hbm, col_v, ew_v, zb_v, deg_s, sem):
    c = lax.axis_index("c")
    s = lax.axis_index("s")
    wid = s * NC + c
    span = NPAD // NS

    def zero_body(i, _):
        zb_v[pl.ds(i * 16, 16)] = jnp.zeros((16,), jnp.float32)
        return 0

    lax.fori_loop(0, span // 16, zero_body, 0)
    pltpu.sync_copy(zb_v, deg_s.at[pl.ds(s * span, span)])
    pltpu.sync_copy(col_hbm.at[wid], col_v)
    pltpu.sync_copy(ew_hbm.at[wid], ew_v)
    plsc.subcore_barrier()

    def fire(j, _):
        pltpu.async_copy(ew_v.at[j], deg_s.at[col_v.at[j]], sem, add=True)
        return 0

    lax.fori_loop(0, NCH, fire, 0)
    pltpu.make_async_copy(ew_hbm.at[wid], ew_v, sem).wait()
    plsc.subcore_barrier()
    pltpu.sync_copy(deg_s.at[pl.ds(s * span, span)],
                    out_hbm.at[c, pl.ds(s * span, span)])


def _make_prop(D):
    span = NPAD // NS

    @functools.partial(
        pl.kernel,
        out_type=jax.ShapeDtypeStruct((NC, NPAD, D), jnp.float32),
        mesh=_mesh(),
        scratch_types=[
            pltpu.VMEM((NCH, CHUNK), jnp.int32),
            pltpu.VMEM((NCH, CHUNK), jnp.int32),
            pltpu.VMEM((NCH, CHUNK), jnp.float32),
            pltpu.VMEM((RING, CHUNK, D), jnp.float32),
            pltpu.VMEM((RING, CHUNK, D), jnp.float32),
            pltpu.VMEM((span, D), jnp.float32),
            pltpu.VMEM_SHARED((NPAD, D), jnp.float32),
        ] + [pltpu.SemaphoreType.DMA] * (2 * RING),
        compiler_params=pltpu.CompilerParams(use_tc_tiling_on_sc=False),
    )
    def prop(row_hbm, col_hbm, ew_hbm, tab_hbm, out_hbm,
             row_v, col_v, ew_v, g_v, s_v, zb_v, agg_s, *sems):
        gsem = sems[:RING]
        ssem = sems[RING:]
        c = lax.axis_index("c")
        s = lax.axis_index("s")
        wid = s * NC + c
        r0 = s * span

        def zero_body(i, _):
            for u in range(D // 16):
                zb_v[i, pl.ds(u * 16, 16)] = jnp.zeros((16,), jnp.float32)
            return 0

        lax.fori_loop(0, span, zero_body, 0)
        pltpu.sync_copy(zb_v, agg_s.at[pl.ds(r0, span), :])
        pltpu.sync_copy(row_hbm.at[wid], row_v)
        pltpu.sync_copy(col_hbm.at[wid], col_v)
        pltpu.sync_copy(ew_hbm.at[wid], ew_v)
        plsc.subcore_barrier()

        def fire_gather(j, b):
            pltpu.async_copy(tab_hbm.at[row_v.at[j]], g_v.at[b], gsem[b])

        def fire_scatter(j, b):
            pltpu.async_copy(s_v.at[b], agg_s.at[col_v.at[j]], ssem[b],
                             add=True)

        def wait_g(b):
            pltpu.make_async_copy(tab_hbm.at[pl.ds(0, CHUNK), :], g_v.at[b],
                                  gsem[b]).wait()

        def wait_s(b):
            pltpu.make_async_copy(tab_hbm.at[pl.ds(0, CHUNK), :], s_v.at[b],
                                  ssem[b]).wait()

        def scale(j, b):
            for g in range(CHUNK // 16):
                w16 = ew_v[j, pl.ds(g * 16, 16)]
                for t in range(16):
                    k = g * 16 + t
                    wb = _bcast_lane(w16, t)
                    for u in range(D // 16):
                        sl = pl.ds(u * 16, 16)
                        s_v[b, k, sl] = g_v[b, k, sl] * wb

        for b in range(RING):
            fire_gather(b, b)

        n_groups = NCH // RING

        def body(i, _):
            for b in range(RING):
                j = i * RING + b
                wait_g(b)

                @pl.when(i >= 1)
                def _():
                    wait_s(b)

                scale(j, b)
                fire_scatter(j, b)

                @pl.when(i < n_groups - 1)
                def _():
                    fire_gather(j + RING, b)

            return 0

        lax.fori_loop(0, n_groups, body, 0)
        for b in range(RING):
            wait_s(b)
        plsc.subcore_barrier()
        pltpu.sync_copy(agg_s.at[pl.ds(r0, span), :],
                        out_hbm.at[c, pl.ds(r0, span), :])

    return prop


_prop16 = _make_prop(HID)
_prop48 = _make_prop(CP)


def _tc1_body(x_ref, w1_ref, dp_ref, xws_ref, dis_ref):
    deg = dp_ref[0] + dp_ref[1]
    dis = jnp.where(deg > 0, lax.rsqrt(deg), 0.0)
    dis_ref[...] = dis
    xw = jnp.dot(x_ref[...], w1_ref[...], preferred_element_type=jnp.float32)
    xws_ref[...] = xw * dis


def _tc2_body(p_ref, dis_ref, b1_ref, w2_ref, hws_ref):
    dis = dis_ref[...]
    agg = (p_ref[0] + p_ref[1]) * dis
    h = jnp.maximum(agg + b1_ref[...], 0.0)
    hw = jnp.dot(h, w2_ref[...], preferred_element_type=jnp.float32)
    hws_ref[...] = hw * dis


def _tc3_body(p_ref, dis_ref, b2_ref, out_ref):
    z = (p_ref[0] + p_ref[1]) * dis_ref[...]
    z = z[:, :C] + b2_ref[...]
    m = jnp.max(z, axis=1, keepdims=True)
    lse = jnp.log(jnp.sum(jnp.exp(z - m), axis=1, keepdims=True)) + m
    out_ref[...] = z - lse


_tc1 = pl.pallas_call(
    _tc1_body,
    out_shape=(jax.ShapeDtypeStruct((N, HID), jnp.float32),
               jax.ShapeDtypeStruct((N, 1), jnp.float32)),
)
_tc2 = pl.pallas_call(
    _tc2_body,
    out_shape=jax.ShapeDtypeStruct((N, CP), jnp.float32),
)
_tc3 = pl.pallas_call(
    _tc3_body,
    out_shape=jax.ShapeDtypeStruct((N, C), jnp.float32),
)


def _pad_slab(a, dtype):
    return jnp.zeros((EPAD,), dtype).at[:E].set(a).reshape(NW, NCH, CHUNK)


def kernel(x, edge_index, edge_attr, W1, b1, W2, b2):
    row = _pad_slab(edge_index[0], jnp.int32)
    col = _pad_slab(edge_index[1], jnp.int32)
    ew = _pad_slab(edge_attr, jnp.float32)
    deg_p = _sc_deg(col, ew)
    xws, dis = _tc1(x, W1, deg_p[:, :N, None])
    p1 = _prop16(row, col, ew, xws)[:, :N]
    w2p = jnp.zeros((HID, CP), jnp.float32).at[:, :C].set(W2)
    hws = _tc2(p1, dis, b1[None, :], w2p)
    p2 = _prop48(row, col, ew, hws)[:, :N]
    return _tc3(p2, dis, b2[None, :])


# speedup vs baseline: 37.9940x; 1.6078x over previous
import functools

import jax
import jax.numpy as jnp
from jax import lax
from jax.experimental import pallas as pl
from jax.experimental.pallas import tpu as pltpu
from jax.experimental.pallas import tpu_sc as plsc

N = 10000
E = 320000
F_IN = 128
HID = 16
C = 40
CP = 48
NPAD = 10240

NC = 2
NS = 16
NW = NC * NS
CHUNK = 80
NCH = 128
EPAD = NW * NCH * CHUNK
RING = 4


def _mesh():
    return plsc.VectorSubcoreMesh(core_axis_name="c", subcore_axis_name="s")


def _bcast_lane(v16, t):
    idx = jnp.full((16, 1), t, jnp.int32)
    dnums = lax.GatherDimensionNumbers(
        offset_dims=(), collapsed_slice_dims=(0,), start_index_map=(0,))
    return lax.gather(v16, idx, dimension_numbers=dnums, slice_sizes=(1,),
                      mode=lax.GatherScatterMode.PROMISE_IN_BOUNDS)


@functools.partial(
    pl.kernel,
    out_type=jax.ShapeDtypeStruct((NC, NPAD), jnp.float32),
    mesh=_mesh(),
    scratch_types=[
        pltpu.VMEM((NCH, CHUNK), jnp.int32),
        pltpu.VMEM((NCH, CHUNK), jnp.float32),
        pltpu.VMEM((NPAD // NS,), jnp.float32),
        pltpu.VMEM_SHARED((NPAD,), jnp.float32),
        pltpu.SemaphoreType.DMA,
    ],
    compiler_params=pltpu.CompilerParams(use_tc_tiling_on_sc=False),
)
def _sc_deg(col_hbm, ew_hbm, out_hbm, col_v, ew_v, zb_v, deg_s, sem):
    c = lax.axis_index("c")
    s = lax.axis_index("s")
    wid = s * NC + c
    span = NPAD // NS

    def zero_body(i, _):
        zb_v[pl.ds(i * 16, 16)] = jnp.zeros((16,), jnp.float32)
        return 0

    lax.fori_loop(0, span // 16, zero_body, 0)
    pltpu.sync_copy(zb_v, deg_s.at[pl.ds(s * span, span)])
    pltpu.sync_copy(col_hbm.at[wid], col_v)
    pltpu.sync_copy(ew_hbm.at[wid], ew_v)
    plsc.subcore_barrier()

    def fire(j, _):
        pltpu.async_copy(ew_v.at[j], deg_s.at[col_v.at[j]], sem, add=True)
        return 0

    lax.fori_loop(0, NCH, fire, 0)
    pltpu.make_async_copy(ew_hbm.at[wid], ew_v, sem).wait()
    plsc.subcore_barrier()
    pltpu.sync_copy(deg_s.at[pl.ds(s * span, span)],
                    out_hbm.at[c, pl.ds(s * span, span)])


def _make_prop(D):
    span = NPAD // NS

    @functools.partial(
        pl.kernel,
        out_type=jax.ShapeDtypeStruct((NC, NPAD, D), jnp.float32),
        mesh=_mesh(),
        scratch_types=[
            pltpu.VMEM((NCH, CHUNK), jnp.int32),
            pltpu.VMEM((NCH, CHUNK), jnp.int32),
            pltpu.VMEM((NCH, CHUNK), jnp.float32),
            pltpu.VMEM((RING, CHUNK, D), jnp.float32),
            pltpu.VMEM((RING, CHUNK, D), jnp.float32),
            pltpu.VMEM((64, D), jnp.float32),
            pltpu.VMEM_SHARED((NPAD, D), jnp.float32),
            pltpu.VMEM_SHARED((N, D), jnp.float32),
        ] + [pltpu.SemaphoreType.DMA] * (2 * RING + 1),
        compiler_params=pltpu.CompilerParams(use_tc_tiling_on_sc=False),
    )
    def prop(row_hbm, col_hbm, ew_hbm, tab_hbm, out_hbm,
             row_v, col_v, ew_v, g_v, s_v, zb_v, agg_s, tab_s, *sems):
        gsem = sems[:RING]
        ssem = sems[RING:2 * RING]
        tsem = sems[2 * RING]
        c = lax.axis_index("c")
        s = lax.axis_index("s")
        wid = s * NC + c
        r0 = s * span

        t0 = s * (N // NS)
        tcopy = pltpu.async_copy(tab_hbm.at[pl.ds(t0, N // NS), :],
                                 tab_s.at[pl.ds(t0, N // NS), :], tsem)

        def zero_body(i, _):
            for u in range(D // 16):
                zb_v[i, pl.ds(u * 16, 16)] = jnp.zeros((16,), jnp.float32)
            return 0

        lax.fori_loop(0, 64, zero_body, 0)

        def zero_out(i, _):
            pltpu.sync_copy(zb_v, agg_s.at[pl.ds(r0 + i * 64, 64), :])
            return 0

        lax.fori_loop(0, span // 64, zero_out, 0)
        pltpu.sync_copy(row_hbm.at[wid], row_v)
        pltpu.sync_copy(col_hbm.at[wid], col_v)
        pltpu.sync_copy(ew_hbm.at[wid], ew_v)
        tcopy.wait()
        plsc.subcore_barrier()

        def fire_gather(j, b):
            pltpu.async_copy(tab_s.at[row_v.at[j]], g_v.at[b], gsem[b])

        def fire_scatter(j, b):
            pltpu.async_copy(s_v.at[b], agg_s.at[col_v.at[j]], ssem[b],
                             add=True)

        def wait_g(b):
            pltpu.make_async_copy(tab_hbm.at[pl.ds(0, CHUNK), :], g_v.at[b],
                                  gsem[b]).wait()

        def wait_s(b):
            pltpu.make_async_copy(tab_hbm.at[pl.ds(0, CHUNK), :], s_v.at[b],
                                  ssem[b]).wait()

        def scale(j, b):
            for g in range(CHUNK // 16):
                w16 = ew_v[j, pl.ds(g * 16, 16)]
                for t in range(16):
                    k = g * 16 + t
                    wb = _bcast_lane(w16, t)
                    for u in range(D // 16):
                        sl = pl.ds(u * 16, 16)
                        s_v[b, k, sl] = g_v[b, k, sl] * wb

        for b in range(RING):
            fire_gather(b, b)

        n_groups = NCH // RING

        def body(i, _):
            for b in range(RING):
                j = i * RING + b
                wait_g(b)

                @pl.when(i >= 1)
                def _():
                    wait_s(b)

                scale(j, b)
                fire_scatter(j, b)

                @pl.when(i < n_groups - 1)
                def _():
                    fire_gather(j + RING, b)

            return 0

        lax.fori_loop(0, n_groups, body, 0)
        for b in range(RING):
            wait_s(b)
        plsc.subcore_barrier()
        pltpu.sync_copy(agg_s.at[pl.ds(r0, span), :],
                        out_hbm.at[c, pl.ds(r0, span), :])

    return prop


_prop16 = _make_prop(HID)
_prop48 = _make_prop(CP)


def _tc1_body(x_ref, w1_ref, dp_ref, xws_ref, dis_ref):
    deg = dp_ref[0] + dp_ref[1]
    dis = jnp.where(deg > 0, lax.rsqrt(deg), 0.0)
    dis_ref[...] = dis
    xw = jnp.dot(x_ref[...], w1_ref[...], preferred_element_type=jnp.float32)
    xws_ref[...] = xw * dis


def _tc2_body(p_ref, dis_ref, b1_ref, w2_ref, hws_ref):
    dis = dis_ref[...]
    agg = (p_ref[0] + p_ref[1]) * dis
    h = jnp.maximum(agg + b1_ref[...], 0.0)
    hw = jnp.dot(h, w2_ref[...], preferred_element_type=jnp.float32)
    hws_ref[...] = hw * dis


def _tc3_body(p_ref, dis_ref, b2_ref, out_ref):
    z = (p_ref[0] + p_ref[1]) * dis_ref[...]
    z = z[:, :C] + b2_ref[...]
    m = jnp.max(z, axis=1, keepdims=True)
    lse = jnp.log(jnp.sum(jnp.exp(z - m), axis=1, keepdims=True)) + m
    out_ref[...] = z - lse


_tc1 = pl.pallas_call(
    _tc1_body,
    out_shape=(jax.ShapeDtypeStruct((N, HID), jnp.float32),
               jax.ShapeDtypeStruct((N, 1), jnp.float32)),
)
_tc2 = pl.pallas_call(
    _tc2_body,
    out_shape=jax.ShapeDtypeStruct((N, CP), jnp.float32),
)
_tc3 = pl.pallas_call(
    _tc3_body,
    out_shape=jax.ShapeDtypeStruct((N, C), jnp.float32),
)


def _pad_slab(a, dtype):
    return jnp.zeros((EPAD,), dtype).at[:E].set(a).reshape(NW, NCH, CHUNK)


def kernel(x, edge_index, edge_attr, W1, b1, W2, b2):
    row = _pad_slab(edge_index[0], jnp.int32)
    col = _pad_slab(edge_index[1], jnp.int32)
    ew = _pad_slab(edge_attr, jnp.float32)
    deg_p = _sc_deg(col, ew)
    xws, dis = _tc1(x, W1, deg_p[:, :N, None])
    p1 = _prop16(row, col, ew, xws)[:, :N]
    w2p = jnp.zeros((HID, CP), jnp.float32).at[:, :C].set(W2)
    hws = _tc2(p1, dis, b1[None, :], w2p)
    p2 = _prop48(row, col, ew, hws)[:, :N]
    return _tc3(p2, dis, b2[None, :])


# speedup vs baseline: 42.7625x; 1.1255x over previous
import functools

import jax
import jax.numpy as jnp
from jax import lax
from jax.experimental import pallas as pl
from jax.experimental.pallas import tpu as pltpu
from jax.experimental.pallas import tpu_sc as plsc

N = 10000
E = 320000
F_IN = 128
HID = 16
C = 40
CP = 48
NPAD = 10240

NC = 2
NS = 16
NW = NC * NS
CHUNK = 80
NCH = 125
RING = 5
SPAN = NPAD // NS
TSPAN = N // NS


def _mesh():
    return plsc.VectorSubcoreMesh(core_axis_name="c", subcore_axis_name="s")


def _bcast_lane(v16, t):
    idx = jnp.full((16, 1), t, jnp.int32)
    dnums = lax.GatherDimensionNumbers(
        offset_dims=(), collapsed_slice_dims=(0,), start_index_map=(0,))
    return lax.gather(v16, idx, dimension_numbers=dnums, slice_sizes=(1,),
                      mode=lax.GatherScatterMode.PROMISE_IN_BOUNDS)


def _rsqrt16(v):
    i = lax.bitcast_convert_type(v, jnp.int32)
    y = lax.bitcast_convert_type(jnp.int32(0x5F3759DF) - (i >> 1), jnp.float32)
    half_v = v * 0.5
    for _ in range(3):
        y = y * (1.5 - half_v * y * y)
    return jnp.where(v > 0.0, y, 0.0)


@functools.partial(
    pl.kernel,
    out_type=(jax.ShapeDtypeStruct((NC, NPAD, HID), jnp.float32),
              jax.ShapeDtypeStruct((NC, NPAD), jnp.float32)),
    mesh=_mesh(),
    scratch_types=[
        pltpu.VMEM((NCH, CHUNK), jnp.int32),
        pltpu.VMEM((NCH, CHUNK), jnp.int32),
        pltpu.VMEM((NCH, CHUNK), jnp.float32),
        pltpu.VMEM((NCH, CHUNK), jnp.int32),
        pltpu.VMEM((NCH, CHUNK), jnp.float32),
        pltpu.VMEM((NCH, CHUNK), jnp.float32),
        pltpu.VMEM((RING, CHUNK, HID), jnp.float32),
        pltpu.VMEM((2, CHUNK, HID), jnp.float32),
        pltpu.VMEM((SPAN,), jnp.float32),
        pltpu.VMEM((64, HID), jnp.float32),
        pltpu.VMEM_SHARED((NPAD,), jnp.float32),
        pltpu.VMEM_SHARED((NPAD,), jnp.float32),
        pltpu.VMEM_SHARED((N, HID), jnp.float32),
        pltpu.VMEM_SHARED((NPAD, HID), jnp.float32),
    ] + [pltpu.SemaphoreType.DMA] * (RING + 2 + 3),
    compiler_params=pltpu.CompilerParams(use_tc_tiling_on_sc=False),
)
def _sc_layer1(row_hbm, col_hbm, ew_hbm, xw_hbm, p1_hbm, dis_hbm,
               row_v, col_v, ew_v, col2_v, ew2_v, pw_v, g_v, s_v, db_v, zb_v,
               deg_s, dis_s, tab_s, agg_s, *sems):
    gsem = sems[:RING]
    ssem = sems[RING:RING + 2]
    tsem, dsem, psem = sems[RING + 2:]
    c = lax.axis_index("c")
    s = lax.axis_index("s")
    wid = s * NC + c
    wid2 = s * NC + (1 - c)
    r0 = s * SPAN
    t0 = s * TSPAN

    tcopy = pltpu.async_copy(xw_hbm.at[pl.ds(t0, TSPAN), :],
                             tab_s.at[pl.ds(t0, TSPAN), :], tsem)

    def zero_db(i, _):
        db_v[pl.ds(i * 16, 16)] = jnp.zeros((16,), jnp.float32)
        return 0

    lax.fori_loop(0, SPAN // 16, zero_db, 0)
    pltpu.sync_copy(db_v, deg_s.at[pl.ds(r0, SPAN)])

    def zero_zb(i, _):
        zb_v[i, :] = jnp.zeros((16,), jnp.float32)
        return 0

    lax.fori_loop(0, 64, zero_zb, 0)

    def zero_agg(i, _):
        pltpu.sync_copy(zb_v, agg_s.at[pl.ds(r0 + i * 64, 64), :])
        return 0

    lax.fori_loop(0, SPAN // 64, zero_agg, 0)

    pltpu.sync_copy(row_hbm.at[wid], row_v)
    pltpu.sync_copy(col_hbm.at[wid], col_v)
    pltpu.sync_copy(ew_hbm.at[wid], ew_v)
    pltpu.sync_copy(col_hbm.at[wid2], col2_v)
    pltpu.sync_copy(ew_hbm.at[wid2], ew2_v)
    plsc.subcore_barrier()

    def fire_deg(j, _):
        pltpu.async_copy(ew_v.at[j], deg_s.at[col_v.at[j]], dsem, add=True)
        pltpu.async_copy(ew2_v.at[j], deg_s.at[col2_v.at[j]], dsem, add=True)
        return 0

    lax.fori_loop(0, NCH, fire_deg, 0)
    pltpu.make_async_copy(ew_hbm.at[wid], ew_v, dsem).wait()
    pltpu.make_async_copy(ew_hbm.at[wid], ew_v, dsem).wait()
    plsc.subcore_barrier()

    pltpu.sync_copy(deg_s.at[pl.ds(r0, SPAN)], db_v)

    def dis_body(i, _):
        sl = pl.ds(i * 16, 16)
        db_v[sl] = _rsqrt16(db_v[sl])
        return 0

    lax.fori_loop(0, SPAN // 16, dis_body, 0)
    pltpu.sync_copy(db_v, dis_s.at[pl.ds(r0, SPAN)])
    pltpu.sync_copy(db_v, dis_hbm.at[c, pl.ds(r0, SPAN)])
    plsc.subcore_barrier()

    def fire_pw(j, _):
        pltpu.async_copy(dis_s.at[row_v.at[j]], pw_v.at[j], psem)
        return 0

    lax.fori_loop(0, NCH, fire_pw, 0)
    pltpu.make_async_copy(ew_hbm.at[wid], pw_v, psem).wait()

    def pw_mul(j, _):
        for g in range(CHUNK // 16):
            sl = pl.ds(g * 16, 16)
            pw_v[j, sl] = pw_v[j, sl] * ew_v[j, sl]
        return 0

    lax.fori_loop(0, NCH, pw_mul, 0)
    tcopy.wait()
    plsc.subcore_barrier()

    def fire_gather(j, b):
        pltpu.async_copy(tab_s.at[row_v.at[j]], g_v.at[b], gsem[b])

    def fire_scatter(j, sb):
        pltpu.async_copy(s_v.at[sb], agg_s.at[col_v.at[j]], ssem[sb],
                         add=True)

    def wait_g(b):
        pltpu.make_async_copy(xw_hbm.at[pl.ds(0, CHUNK), :], g_v.at[b],
                              gsem[b]).wait()

    def wait_s(sb):
        pltpu.make_async_copy(xw_hbm.at[pl.ds(0, CHUNK), :], s_v.at[sb],
                              ssem[sb]).wait()

    def scale(j, b, sb):
        for g in range(CHUNK // 16):
            w16 = pw_v[j, pl.ds(g * 16, 16)]
            for t in range(16):
                k = g * 16 + t
                wb = _bcast_lane(w16, t)
                s_v[sb, k, :] = g_v[b, k, :] * wb

    for b in range(RING):
        fire_gather(b, b)

    n_groups = NCH // RING

    def body(i, _):
        for b in range(RING):
            j = i * RING + b
            sb = b % 2
            wait_g(b)

            if b < 2:
                @pl.when(i >= 1)
                def _():
                    wait_s(sb)
            else:
                wait_s(sb)

            scale(j, b, sb)
            fire_scatter(j, sb)

            @pl.when(i < n_groups - 1)
            def _():
                fire_gather(j + RING, b)

        return 0

    lax.fori_loop(0, n_groups, body, 0)
    for sb in range(2):
        wait_s(sb)
    plsc.subcore_barrier()
    pltpu.sync_copy(agg_s.at[pl.ds(r0, SPAN), :],
                    p1_hbm.at[c, pl.ds(r0, SPAN), :])


@functools.partial(
    pl.kernel,
    out_type=jax.ShapeDtypeStruct((NC, NPAD, CP), jnp.float32),
    mesh=_mesh(),
    scratch_types=[
        pltpu.VMEM((NCH, CHUNK), jnp.int32),
        pltpu.VMEM((NCH, CHUNK), jnp.int32),
        pltpu.VMEM((NCH, CHUNK), jnp.float32),
        pltpu.VMEM((RING, CHUNK, CP), jnp.float32),
        pltpu.VMEM((2, CHUNK, CP), jnp.float32),
        pltpu.VMEM((64, CP), jnp.float32),
        pltpu.VMEM_SHARED((NPAD, CP), jnp.float32),
        pltpu.VMEM_SHARED((N, CP), jnp.float32),
    ] + [pltpu.SemaphoreType.DMA] * (RING + 2 + 1),
    compiler_params=pltpu.CompilerParams(use_tc_tiling_on_sc=False),
)
def _sc_layer2(row_hbm, col_hbm, ew_hbm, tab_hbm, out_hbm,
               row_v, col_v, ew_v, g_v, s_v, zb_v, agg_s, tab_s, *sems):
    gsem = sems[:RING]
    ssem = sems[RING:RING + 2]
    tsem = sems[RING + 2]
    c = lax.axis_index("c")
    s = lax.axis_index("s")
    wid = s * NC + c
    r0 = s * SPAN
    t0 = s * TSPAN

    tcopy = pltpu.async_copy(tab_hbm.at[pl.ds(t0, TSPAN), :],
                             tab_s.at[pl.ds(t0, TSPAN), :], tsem)

    def zero_zb(i, _):
        for u in range(CP // 16):
            zb_v[i, pl.ds(u * 16, 16)] = jnp.zeros((16,), jnp.float32)
        return 0

    lax.fori_loop(0, 64, zero_zb, 0)

    def zero_agg(i, _):
        pltpu.sync_copy(zb_v, agg_s.at[pl.ds(r0 + i * 64, 64), :])
        return 0

    lax.fori_loop(0, SPAN // 64, zero_agg, 0)

    pltpu.sync_copy(row_hbm.at[wid], row_v)
    pltpu.sync_copy(col_hbm.at[wid], col_v)
    pltpu.sync_copy(ew_hbm.at[wid], ew_v)
    tcopy.wait()
    plsc.subcore_barrier()

    def fire_gather(j, b):
        pltpu.async_copy(tab_s.at[row_v.at[j]], g_v.at[b], gsem[b])

    def fire_scatter(j, sb):
        pltpu.async_copy(s_v.at[sb], agg_s.at[col_v.at[j]], ssem[sb],
                         add=True)

    def wait_g(b):
        pltpu.make_async_copy(tab_hbm.at[pl.ds(0, CHUNK), :], g_v.at[b],
                              gsem[b]).wait()

    def wait_s(sb):
        pltpu.make_async_copy(tab_hbm.at[pl.ds(0, CHUNK), :], s_v.at[sb],
                              ssem[sb]).wait()

    def scale(j, b, sb):
        for g in range(CHUNK // 16):
            w16 = ew_v[j, pl.ds(g * 16, 16)]
            for t in range(16):
                k = g * 16 + t
                wb = _bcast_lane(w16, t)
                for u in range(CP // 16):
                    sl = pl.ds(u * 16, 16)
                    s_v[sb, k, sl] = g_v[b, k, sl] * wb

    for b in range(RING):
        fire_gather(b, b)

    n_groups = NCH // RING

    def body(i, _):
        for b in range(RING):
            j = i * RING + b
            sb = b % 2
            wait_g(b)

            if b < 2:
                @pl.when(i >= 1)
                def _():
                    wait_s(sb)
            else:
                wait_s(sb)

            scale(j, b, sb)
            fire_scatter(j, sb)

            @pl.when(i < n_groups - 1)
            def _():
                fire_gather(j + RING, b)

        return 0

    lax.fori_loop(0, n_groups, body, 0)
    for sb in range(2):
        wait_s(sb)
    plsc.subcore_barrier()
    pltpu.sync_copy(agg_s.at[pl.ds(r0, SPAN), :],
                    out_hbm.at[c, pl.ds(r0, SPAN), :])


def _tc1_body(x_ref, w1_ref, xw_ref):
    xw_ref[...] = jnp.dot(x_ref[...], w1_ref[...],
                          preferred_element_type=jnp.float32)


def _tc2_body(p_ref, dis_ref, b1_ref, w2_ref, hws_ref):
    dis = dis_ref[...][0, :N, None]
    p = p_ref[...]
    agg = (p[0, :N] + p[1, :N]) * dis
    h = jnp.maximum(agg + b1_ref[...], 0.0)
    hw = jnp.dot(h, w2_ref[...], preferred_element_type=jnp.float32)
    hws_ref[...] = hw * dis


def _tc3_body(p_ref, dis_ref, b2_ref, out_ref):
    p = p_ref[...]
    z = (p[0, :N] + p[1, :N]) * dis_ref[...][0, :N, None]
    z = z[:, :C] + b2_ref[...]
    m = jnp.max(z, axis=1, keepdims=True)
    lse = jnp.log(jnp.sum(jnp.exp(z - m), axis=1, keepdims=True)) + m
    out_ref[...] = z - lse


_tc1 = pl.pallas_call(
    _tc1_body,
    out_shape=jax.ShapeDtypeStruct((N, HID), jnp.float32),
)
_tc2 = pl.pallas_call(
    _tc2_body,
    out_shape=jax.ShapeDtypeStruct((N, CP), jnp.float32),
)
_tc3 = pl.pallas_call(
    _tc3_body,
    out_shape=jax.ShapeDtypeStruct((N, C), jnp.float32),
)


def kernel(x, edge_index, edge_attr, W1, b1, W2, b2):
    row = edge_index[0].reshape(NW, NCH, CHUNK)
    col = edge_index[1].reshape(NW, NCH, CHUNK)
    ew = edge_attr.reshape(NW, NCH, CHUNK)
    xw = _tc1(x, W1)
    p1, dis = _sc_layer1(row, col, ew, xw)
    w2p = jnp.zeros((HID, CP), jnp.float32).at[:, :C].set(W2)
    hws = _tc2(p1, dis, b1[None, :], w2p)
    p2 = _sc_layer2(row, col, ew, hws)
    return _tc3(p2, dis, b2[None, :])


# speedup vs baseline: 55.2003x; 1.2909x over previous
import functools

import jax
import jax.numpy as jnp
from jax import lax
from jax.experimental import pallas as pl
from jax.experimental.pallas import tpu as pltpu
from jax.experimental.pallas import tpu_sc as plsc

N = 10000
E = 320000
F_IN = 128
HID = 16
C = 40
CP = 48
NPAD = 10240

NC = 2
NS = 16
NW = NC * NS
CHUNK = 80
NCH = 125
RING = 5
SPAN = NPAD // NS
TSPAN = N // NS


def _mesh():
    return plsc.VectorSubcoreMesh(core_axis_name="c", subcore_axis_name="s")


def _bcast_lane(v16, t):
    idx = jnp.full((16, 1), t, jnp.int32)
    dnums = lax.GatherDimensionNumbers(
        offset_dims=(), collapsed_slice_dims=(0,), start_index_map=(0,))
    return lax.gather(v16, idx, dimension_numbers=dnums, slice_sizes=(1,),
                      mode=lax.GatherScatterMode.PROMISE_IN_BOUNDS)


def _rsqrt16(v):
    i = lax.bitcast_convert_type(v, jnp.int32)
    y = lax.bitcast_convert_type(jnp.int32(0x5F3759DF) - (i >> 1), jnp.float32)
    half_v = v * 0.5
    for _ in range(3):
        y = y * (1.5 - half_v * y * y)
    return jnp.where(v > 0.0, y, 0.0)


@functools.partial(
    pl.kernel,
    out_type=(jax.ShapeDtypeStruct((NC, NPAD, HID), jnp.float32),
              jax.ShapeDtypeStruct((NC, NPAD), jnp.float32),
              jax.ShapeDtypeStruct((NW, NCH, CHUNK), jnp.float32)),
    mesh=_mesh(),
    scratch_types=[
        pltpu.VMEM((NCH, CHUNK), jnp.int32),
        pltpu.VMEM((NCH, CHUNK), jnp.int32),
        pltpu.VMEM((NCH, CHUNK), jnp.float32),
        pltpu.VMEM((NCH, CHUNK), jnp.int32),
        pltpu.VMEM((NCH, CHUNK), jnp.float32),
        pltpu.VMEM((NCH, CHUNK), jnp.float32),
        pltpu.VMEM((RING, CHUNK, HID), jnp.float32),
        pltpu.VMEM((2, CHUNK, HID), jnp.float32),
        pltpu.VMEM((SPAN,), jnp.float32),
        pltpu.VMEM((64, HID), jnp.float32),
        pltpu.VMEM_SHARED((NPAD,), jnp.float32),
        pltpu.VMEM_SHARED((NPAD,), jnp.float32),
        pltpu.VMEM_SHARED((N, HID), jnp.float32),
        pltpu.VMEM_SHARED((NPAD, HID), jnp.float32),
    ] + [pltpu.SemaphoreType.DMA] * (RING + 2 + 3),
    compiler_params=pltpu.CompilerParams(use_tc_tiling_on_sc=False,
                                         disable_bounds_checks=True),
)
def _sc_layer1(row_hbm, col_hbm, ew_hbm, xw_hbm, p1_hbm, dis_hbm, pw_hbm,
               row_v, col_v, ew_v, col2_v, ew2_v, pw_v, g_v, s_v, db_v, zb_v,
               deg_s, dis_s, tab_s, agg_s, *sems):
    gsem = sems[:RING]
    ssem = sems[RING:RING + 2]
    tsem, dsem, psem = sems[RING + 2:]
    c = lax.axis_index("c")
    s = lax.axis_index("s")
    wid = s * NC + c
    wid2 = s * NC + (1 - c)
    r0 = s * SPAN
    t0 = s * TSPAN

    tcopy = pltpu.async_copy(xw_hbm.at[pl.ds(t0, TSPAN), :],
                             tab_s.at[pl.ds(t0, TSPAN), :], tsem)

    def zero_db(i, _):
        db_v[pl.ds(i * 16, 16)] = jnp.zeros((16,), jnp.float32)
        return 0

    lax.fori_loop(0, SPAN // 16, zero_db, 0)
    pltpu.sync_copy(db_v, deg_s.at[pl.ds(r0, SPAN)])

    def zero_zb(i, _):
        zb_v[i, :] = jnp.zeros((16,), jnp.float32)
        return 0

    lax.fori_loop(0, 64, zero_zb, 0)

    def zero_agg(i, _):
        pltpu.sync_copy(zb_v, agg_s.at[pl.ds(r0 + i * 64, 64), :])
        return 0

    lax.fori_loop(0, SPAN // 64, zero_agg, 0)

    pltpu.sync_copy(row_hbm.at[wid], row_v)
    pltpu.sync_copy(col_hbm.at[wid], col_v)
    pltpu.sync_copy(ew_hbm.at[wid], ew_v)
    pltpu.sync_copy(col_hbm.at[wid2], col2_v)
    pltpu.sync_copy(ew_hbm.at[wid2], ew2_v)
    plsc.subcore_barrier()

    def fire_deg(j, _):
        pltpu.async_copy(ew_v.at[j], deg_s.at[col_v.at[j]], dsem, add=True)
        pltpu.async_copy(ew2_v.at[j], deg_s.at[col2_v.at[j]], dsem, add=True)
        return 0

    lax.fori_loop(0, NCH, fire_deg, 0)
    pltpu.make_async_copy(ew_hbm.at[wid], ew_v, dsem).wait()
    pltpu.make_async_copy(ew_hbm.at[wid], ew_v, dsem).wait()
    plsc.subcore_barrier()

    pltpu.sync_copy(deg_s.at[pl.ds(r0, SPAN)], db_v)

    def dis_body(i, _):
        sl = pl.ds(i * 16, 16)
        db_v[sl] = _rsqrt16(db_v[sl])
        return 0

    lax.fori_loop(0, SPAN // 16, dis_body, 0)
    pltpu.sync_copy(db_v, dis_s.at[pl.ds(r0, SPAN)])
    pltpu.sync_copy(db_v, dis_hbm.at[c, pl.ds(r0, SPAN)])
    plsc.subcore_barrier()

    def fire_pw(j, _):
        pltpu.async_copy(dis_s.at[row_v.at[j]], pw_v.at[j], psem)
        return 0

    lax.fori_loop(0, NCH, fire_pw, 0)
    pltpu.make_async_copy(ew_hbm.at[wid], pw_v, psem).wait()

    def pw_mul(j, _):
        for g in range(CHUNK // 16):
            sl = pl.ds(g * 16, 16)
            pw_v[j, sl] = pw_v[j, sl] * ew_v[j, sl]
        return 0

    lax.fori_loop(0, NCH, pw_mul, 0)
    pltpu.sync_copy(pw_v, pw_hbm.at[wid])
    tcopy.wait()
    plsc.subcore_barrier()

    def fire_gather(j, b):
        pltpu.async_copy(tab_s.at[row_v.at[j]], g_v.at[b], gsem[b])

    def fire_scatter(j, sb):
        pltpu.async_copy(s_v.at[sb], agg_s.at[col_v.at[j]], ssem[sb],
                         add=True)

    def wait_g(b):
        pltpu.make_async_copy(xw_hbm.at[pl.ds(0, CHUNK), :], g_v.at[b],
                              gsem[b]).wait()

    def wait_s(sb):
        pltpu.make_async_copy(xw_hbm.at[pl.ds(0, CHUNK), :], s_v.at[sb],
                              ssem[sb]).wait()

    def scale(j, b, sb):
        for g in range(CHUNK // 16):
            w16 = pw_v[j, pl.ds(g * 16, 16)]
            for t in range(16):
                k = g * 16 + t
                wb = _bcast_lane(w16, t)
                s_v[sb, k, :] = g_v[b, k, :] * wb

    for b in range(RING):
        fire_gather(b, b)

    n_groups = NCH // RING

    def body(i, _):
        for b in range(RING):
            j = i * RING + b
            sb = b % 2
            wait_g(b)

            if b < 2:
                @pl.when(i >= 1)
                def _():
                    wait_s(sb)
            else:
                wait_s(sb)

            scale(j, b, sb)
            fire_scatter(j, sb)

            @pl.when(i < n_groups - 1)
            def _():
                fire_gather(j + RING, b)

        return 0

    lax.fori_loop(0, n_groups, body, 0)
    for sb in range(2):
        wait_s(sb)
    plsc.subcore_barrier()
    pltpu.sync_copy(agg_s.at[pl.ds(r0, SPAN), :],
                    p1_hbm.at[c, pl.ds(r0, SPAN), :])


@functools.partial(
    pl.kernel,
    out_type=jax.ShapeDtypeStruct((NC, NPAD, HID), jnp.float32),
    mesh=_mesh(),
    scratch_types=[
        pltpu.VMEM((NCH, CHUNK), jnp.int32),
        pltpu.VMEM((NCH, CHUNK), jnp.int32),
        pltpu.VMEM((NCH, CHUNK), jnp.float32),
        pltpu.VMEM((RING, CHUNK, HID), jnp.float32),
        pltpu.VMEM((2, CHUNK, HID), jnp.float32),
        pltpu.VMEM((64, HID), jnp.float32),
        pltpu.VMEM_SHARED((NPAD, HID), jnp.float32),
        pltpu.VMEM_SHARED((N, HID), jnp.float32),
    ] + [pltpu.SemaphoreType.DMA] * (RING + 2 + 1),
    compiler_params=pltpu.CompilerParams(use_tc_tiling_on_sc=False,
                                         disable_bounds_checks=True),
)
def _sc_layer2(row_hbm, col_hbm, ew_hbm, tab_hbm, out_hbm,
               row_v, col_v, ew_v, g_v, s_v, zb_v, agg_s, tab_s, *sems):
    gsem = sems[:RING]
    ssem = sems[RING:RING + 2]
    tsem = sems[RING + 2]
    c = lax.axis_index("c")
    s = lax.axis_index("s")
    wid = s * NC + c
    r0 = s * SPAN
    t0 = s * TSPAN

    tcopy = pltpu.async_copy(tab_hbm.at[pl.ds(t0, TSPAN), :],
                             tab_s.at[pl.ds(t0, TSPAN), :], tsem)

    def zero_zb(i, _):
        zb_v[i, :] = jnp.zeros((16,), jnp.float32)
        return 0

    lax.fori_loop(0, 64, zero_zb, 0)

    def zero_agg(i, _):
        pltpu.sync_copy(zb_v, agg_s.at[pl.ds(r0 + i * 64, 64), :])
        return 0

    lax.fori_loop(0, SPAN // 64, zero_agg, 0)

    pltpu.sync_copy(row_hbm.at[wid], row_v)
    pltpu.sync_copy(col_hbm.at[wid], col_v)
    pltpu.sync_copy(ew_hbm.at[wid], ew_v)
    tcopy.wait()
    plsc.subcore_barrier()

    def fire_gather(j, b):
        pltpu.async_copy(tab_s.at[row_v.at[j]], g_v.at[b], gsem[b])

    def fire_scatter(j, sb):
        pltpu.async_copy(s_v.at[sb], agg_s.at[col_v.at[j]], ssem[sb],
                         add=True)

    def wait_g(b):
        pltpu.make_async_copy(tab_hbm.at[pl.ds(0, CHUNK), :], g_v.at[b],
                              gsem[b]).wait()

    def wait_s(sb):
        pltpu.make_async_copy(tab_hbm.at[pl.ds(0, CHUNK), :], s_v.at[sb],
                              ssem[sb]).wait()

    def scale(j, b, sb):
        for g in range(CHUNK // 16):
            w16 = ew_v[j, pl.ds(g * 16, 16)]
            for t in range(16):
                k = g * 16 + t
                wb = _bcast_lane(w16, t)
                s_v[sb, k, :] = g_v[b, k, :] * wb

    for b in range(RING):
        fire_gather(b, b)

    n_groups = NCH // RING

    def body(i, _):
        for b in range(RING):
            j = i * RING + b
            sb = b % 2
            wait_g(b)

            if b < 2:
                @pl.when(i >= 1)
                def _():
                    wait_s(sb)
            else:
                wait_s(sb)

            scale(j, b, sb)
            fire_scatter(j, sb)

            @pl.when(i < n_groups - 1)
            def _():
                fire_gather(j + RING, b)

        return 0

    lax.fori_loop(0, n_groups, body, 0)
    for sb in range(2):
        wait_s(sb)
    plsc.subcore_barrier()
    pltpu.sync_copy(agg_s.at[pl.ds(r0, SPAN), :],
                    out_hbm.at[c, pl.ds(r0, SPAN), :])


def _tc1_body(x_ref, w1_ref, xw_ref):
    xw_ref[...] = jnp.dot(x_ref[...], w1_ref[...],
                          preferred_element_type=jnp.float32)


def _tc2_body(p_ref, dis_ref, b1_ref, h_ref):
    dis = dis_ref[...][0, :N, None]
    p = p_ref[...]
    agg = (p[0, :N] + p[1, :N]) * dis
    h_ref[...] = jnp.maximum(agg + b1_ref[...], 0.0)


def _tc3_body(p_ref, dis_ref, w2_ref, b2_ref, out_ref):
    p = p_ref[...]
    q = (p[0, :N] + p[1, :N]) * dis_ref[...][0, :N, None]
    z = jnp.dot(q, w2_ref[...], preferred_element_type=jnp.float32)
    z = z + b2_ref[...]
    m = jnp.max(z, axis=1, keepdims=True)
    lse = jnp.log(jnp.sum(jnp.exp(z - m), axis=1, keepdims=True)) + m
    out_ref[...] = z - lse


_tc1 = pl.pallas_call(
    _tc1_body,
    out_shape=jax.ShapeDtypeStruct((N, HID), jnp.float32),
)
_tc2 = pl.pallas_call(
    _tc2_body,
    out_shape=jax.ShapeDtypeStruct((N, HID), jnp.float32),
)
_tc3 = pl.pallas_call(
    _tc3_body,
    out_shape=jax.ShapeDtypeStruct((N, C), jnp.float32),
)


def kernel(x, edge_index, edge_attr, W1, b1, W2, b2):
    row = edge_index[0].reshape(NW, NCH, CHUNK)
    col = edge_index[1].reshape(NW, NCH, CHUNK)
    ew = edge_attr.reshape(NW, NCH, CHUNK)
    xw = _tc1(x, W1)
    p1, dis, pw = _sc_layer1(row, col, ew, xw)
    h = _tc2(p1, dis, b1[None, :])
    p2 = _sc_layer2(row, col, pw, h)
    return _tc3(p2, dis, W2, b2[None, :])


# speedup vs baseline: 67.3963x; 1.2209x over previous
import functools

import jax
import jax.numpy as jnp
from jax import lax
from jax.experimental import pallas as pl
from jax.experimental.pallas import tpu as pltpu
from jax.experimental.pallas import tpu_sc as plsc

N = 10000
E = 320000
F_IN = 128
HID = 16
C = 40
CP = 48
NPAD = 10240

NC = 2
NS = 16
NW = NC * NS
CHUNK = 80
NCH = 125
RING = 5
SPAN = NPAD // NS
TSPAN = N // NS


def _mesh():
    return plsc.VectorSubcoreMesh(core_axis_name="c", subcore_axis_name="s")


def _bcast_lane(v16, t):
    idx = jnp.full((16, 1), t, jnp.int32)
    dnums = lax.GatherDimensionNumbers(
        offset_dims=(), collapsed_slice_dims=(0,), start_index_map=(0,))
    return lax.gather(v16, idx, dimension_numbers=dnums, slice_sizes=(1,),
                      mode=lax.GatherScatterMode.PROMISE_IN_BOUNDS)


def _rsqrt16(v):
    i = lax.bitcast_convert_type(v, jnp.int32)
    y = lax.bitcast_convert_type(jnp.int32(0x5F3759DF) - (i >> 1), jnp.float32)
    half_v = v * 0.5
    for _ in range(3):
        y = y * (1.5 - half_v * y * y)
    return jnp.where(v > 0.0, y, 0.0)


@functools.partial(
    pl.kernel,
    out_type=(jax.ShapeDtypeStruct((NC, NPAD, HID), jnp.float32),
              jax.ShapeDtypeStruct((NC, NPAD), jnp.float32),
              jax.ShapeDtypeStruct((NW, NCH, CHUNK), jnp.float32)),
    mesh=_mesh(),
    scratch_types=[
        pltpu.VMEM((NCH, CHUNK), jnp.int32),
        pltpu.VMEM((NCH, CHUNK), jnp.int32),
        pltpu.VMEM((NCH, CHUNK), jnp.float32),
        pltpu.VMEM((NCH, CHUNK), jnp.int32),
        pltpu.VMEM((NCH, CHUNK), jnp.float32),
        pltpu.VMEM((NCH, CHUNK), jnp.float32),
        pltpu.VMEM((RING, CHUNK, HID), jnp.float32),
        pltpu.VMEM((2, CHUNK, HID), jnp.float32),
        pltpu.VMEM((SPAN,), jnp.float32),
        pltpu.VMEM((SPAN, HID), jnp.float32),
        pltpu.VMEM((64, HID), jnp.float32),
        pltpu.VMEM_SHARED((NPAD,), jnp.float32),
        pltpu.VMEM_SHARED((NPAD,), jnp.float32),
        pltpu.VMEM_SHARED((N, HID), jnp.float32),
        pltpu.VMEM_SHARED((NPAD, HID), jnp.float32),
    ] + [pltpu.SemaphoreType.DMA] * (RING + 2 + 3),
    compiler_params=pltpu.CompilerParams(use_tc_tiling_on_sc=False,
                                         disable_bounds_checks=True),
)
def _sc_layer1(ei_hbm, ew_hbm, xw_hbm, p1_hbm, dis_hbm, pw_hbm,
               row_v, col_v, ew_v, col2_v, ew2_v, pw_v, g_v, s_v, db_v, ob_v,
               zb_v, deg_s, dis_s, tab_s, agg_s, *sems):
    gsem = sems[:RING]
    ssem = sems[RING:RING + 2]
    tsem, dsem, psem = sems[RING + 2:]
    c = lax.axis_index("c")
    s = lax.axis_index("s")
    wid = s * NC + c
    wid2 = s * NC + (1 - c)
    r0 = s * SPAN
    t0 = s * TSPAN

    tcopy = pltpu.async_copy(xw_hbm.at[pl.ds(t0, TSPAN), :],
                             tab_s.at[pl.ds(t0, TSPAN), :], tsem)

    def zero_db(i, _):
        db_v[pl.ds(i * 16, 16)] = jnp.zeros((16,), jnp.float32)
        return 0

    lax.fori_loop(0, SPAN // 16, zero_db, 0)
    pltpu.sync_copy(db_v, deg_s.at[pl.ds(r0, SPAN)])

    def zero_zb(i, _):
        zb_v[i, :] = jnp.zeros((16,), jnp.float32)
        return 0

    lax.fori_loop(0, 64, zero_zb, 0)

    def zero_agg(i, _):
        pltpu.sync_copy(zb_v, agg_s.at[pl.ds(r0 + i * 64, 64), :])
        return 0

    lax.fori_loop(0, SPAN // 64, zero_agg, 0)

    pltpu.sync_copy(ei_hbm.at[0, wid], row_v)
    pltpu.sync_copy(ei_hbm.at[1, wid], col_v)
    pltpu.sync_copy(ew_hbm.at[wid], ew_v)
    pltpu.sync_copy(ei_hbm.at[1, wid2], col2_v)
    pltpu.sync_copy(ew_hbm.at[wid2], ew2_v)
    plsc.subcore_barrier()

    def fire_deg(j, _):
        pltpu.async_copy(ew_v.at[j], deg_s.at[col_v.at[j]], dsem, add=True)
        pltpu.async_copy(ew2_v.at[j], deg_s.at[col2_v.at[j]], dsem, add=True)
        return 0

    lax.fori_loop(0, NCH, fire_deg, 0)
    pltpu.make_async_copy(ew_hbm.at[wid], ew_v, dsem).wait()
    pltpu.make_async_copy(ew_hbm.at[wid], ew_v, dsem).wait()
    plsc.subcore_barrier()

    pltpu.sync_copy(deg_s.at[pl.ds(r0, SPAN)], db_v)

    def dis_body(i, _):
        sl = pl.ds(i * 16, 16)
        db_v[sl] = _rsqrt16(db_v[sl])
        return 0

    lax.fori_loop(0, SPAN // 16, dis_body, 0)
    pltpu.sync_copy(db_v, dis_s.at[pl.ds(r0, SPAN)])
    pltpu.sync_copy(db_v, dis_hbm.at[c, pl.ds(r0, SPAN)])
    plsc.subcore_barrier()

    def fire_pw(j, _):
        pltpu.async_copy(dis_s.at[row_v.at[j]], pw_v.at[j], psem)
        return 0

    lax.fori_loop(0, NCH, fire_pw, 0)
    pltpu.make_async_copy(ew_hbm.at[wid], pw_v, psem).wait()

    def pw_mul(j, _):
        for g in range(CHUNK // 16):
            sl = pl.ds(g * 16, 16)
            pw_v[j, sl] = pw_v[j, sl] * ew_v[j, sl]
        return 0

    lax.fori_loop(0, NCH, pw_mul, 0)
    pltpu.sync_copy(pw_v, pw_hbm.at[wid])
    tcopy.wait()
    plsc.subcore_barrier()

    def fire_gather(j, b):
        pltpu.async_copy(tab_s.at[row_v.at[j]], g_v.at[b], gsem[b])

    def fire_scatter(j, sb):
        pltpu.async_copy(s_v.at[sb], agg_s.at[col_v.at[j]], ssem[sb],
                         add=True)

    def wait_g(b):
        pltpu.make_async_copy(xw_hbm.at[pl.ds(0, CHUNK), :], g_v.at[b],
                              gsem[b]).wait()

    def wait_s(sb):
        pltpu.make_async_copy(xw_hbm.at[pl.ds(0, CHUNK), :], s_v.at[sb],
                              ssem[sb]).wait()

    def scale(j, b, sb):
        for g in range(CHUNK // 16):
            w16 = pw_v[j, pl.ds(g * 16, 16)]
            for t in range(16):
                k = g * 16 + t
                wb = _bcast_lane(w16, t)
                s_v[sb, k, :] = g_v[b, k, :] * wb

    for b in range(RING):
        fire_gather(b, b)

    n_groups = NCH // RING

    def body(i, _):
        for b in range(RING):
            j = i * RING + b
            sb = b % 2
            wait_g(b)

            if b < 2:
                @pl.when(i >= 1)
                def _():
                    wait_s(sb)
            else:
                wait_s(sb)

            scale(j, b, sb)
            fire_scatter(j, sb)

            @pl.when(i < n_groups - 1)
            def _():
                fire_gather(j + RING, b)

        return 0

    lax.fori_loop(0, n_groups, body, 0)
    for sb in range(2):
        wait_s(sb)
    plsc.subcore_barrier()
    pltpu.sync_copy(agg_s.at[pl.ds(r0, SPAN), :], ob_v)

    def oscale(g, _):
        dv16 = db_v[pl.ds(g * 16, 16)]
        for t in range(16):
            n = g * 16 + t
            wb = _bcast_lane(dv16, t)
            ob_v[n, :] = ob_v[n, :] * wb
        return 0

    lax.fori_loop(0, SPAN // 16, oscale, 0)
    pltpu.sync_copy(ob_v, p1_hbm.at[c, pl.ds(r0, SPAN), :])


@functools.partial(
    pl.kernel,
    out_type=jax.ShapeDtypeStruct((NC, NPAD, HID), jnp.float32),
    mesh=_mesh(),
    scratch_types=[
        pltpu.VMEM((NCH, CHUNK), jnp.int32),
        pltpu.VMEM((NCH, CHUNK), jnp.int32),
        pltpu.VMEM((NCH, CHUNK), jnp.float32),
        pltpu.VMEM((RING, CHUNK, HID), jnp.float32),
        pltpu.VMEM((2, CHUNK, HID), jnp.float32),
        pltpu.VMEM((SPAN,), jnp.float32),
        pltpu.VMEM((SPAN, HID), jnp.float32),
        pltpu.VMEM((64, HID), jnp.float32),
        pltpu.VMEM_SHARED((NPAD, HID), jnp.float32),
        pltpu.VMEM_SHARED((NPAD, HID), jnp.float32),
    ] + [pltpu.SemaphoreType.DMA] * (RING + 2 + 1),
    compiler_params=pltpu.CompilerParams(use_tc_tiling_on_sc=False,
                                         disable_bounds_checks=True),
)
def _sc_layer2(ei_hbm, ew_hbm, tab_hbm, dis_hbm, out_hbm,
               row_v, col_v, ew_v, g_v, s_v, db_v, ob_v, zb_v, agg_s, tab_s,
               *sems):
    gsem = sems[:RING]
    ssem = sems[RING:RING + 2]
    tsem = sems[RING + 2]
    c = lax.axis_index("c")
    s = lax.axis_index("s")
    wid = s * NC + c
    r0 = s * SPAN

    tcopy = pltpu.async_copy(tab_hbm.at[pl.ds(r0, SPAN), :],
                             tab_s.at[pl.ds(r0, SPAN), :], tsem)

    def zero_zb(i, _):
        zb_v[i, :] = jnp.zeros((16,), jnp.float32)
        return 0

    lax.fori_loop(0, 64, zero_zb, 0)

    def zero_agg(i, _):
        pltpu.sync_copy(zb_v, agg_s.at[pl.ds(r0 + i * 64, 64), :])
        return 0

    lax.fori_loop(0, SPAN // 64, zero_agg, 0)

    pltpu.sync_copy(ei_hbm.at[0, wid], row_v)
    pltpu.sync_copy(ei_hbm.at[1, wid], col_v)
    pltpu.sync_copy(ew_hbm.at[wid], ew_v)
    pltpu.sync_copy(dis_hbm.at[c, pl.ds(r0, SPAN)], db_v)
    tcopy.wait()
    plsc.subcore_barrier()

    def fire_gather(j, b):
        pltpu.async_copy(tab_s.at[row_v.at[j]], g_v.at[b], gsem[b])

    def fire_scatter(j, sb):
        pltpu.async_copy(s_v.at[sb], agg_s.at[col_v.at[j]], ssem[sb],
                         add=True)

    def wait_g(b):
        pltpu.make_async_copy(tab_hbm.at[pl.ds(0, CHUNK), :], g_v.at[b],
                              gsem[b]).wait()

    def wait_s(sb):
        pltpu.make_async_copy(tab_hbm.at[pl.ds(0, CHUNK), :], s_v.at[sb],
                              ssem[sb]).wait()

    def scale(j, b, sb):
        for g in range(CHUNK // 16):
            w16 = ew_v[j, pl.ds(g * 16, 16)]
            for t in range(16):
                k = g * 16 + t
                wb = _bcast_lane(w16, t)
                s_v[sb, k, :] = g_v[b, k, :] * wb

    for b in range(RING):
        fire_gather(b, b)

    n_groups = NCH // RING

    def body(i, _):
        for b in range(RING):
            j = i * RING + b
            sb = b % 2
            wait_g(b)

            if b < 2:
                @pl.when(i >= 1)
                def _():
                    wait_s(sb)
            else:
                wait_s(sb)

            scale(j, b, sb)
            fire_scatter(j, sb)

            @pl.when(i < n_groups - 1)
            def _():
                fire_gather(j + RING, b)

        return 0

    lax.fori_loop(0, n_groups, body, 0)
    for sb in range(2):
        wait_s(sb)
    plsc.subcore_barrier()
    pltpu.sync_copy(agg_s.at[pl.ds(r0, SPAN), :], ob_v)

    def oscale(g, _):
        dv16 = db_v[pl.ds(g * 16, 16)]
        for t in range(16):
            n = g * 16 + t
            wb = _bcast_lane(dv16, t)
            ob_v[n, :] = ob_v[n, :] * wb
        return 0

    lax.fori_loop(0, SPAN // 16, oscale, 0)
    pltpu.sync_copy(ob_v, out_hbm.at[c, pl.ds(r0, SPAN), :])


def _tc1_body(x_ref, w1_ref, xw_ref):
    xw_ref[...] = jnp.dot(x_ref[...], w1_ref[...],
                          preferred_element_type=jnp.float32)


def _tc2_body(p_ref, b1_ref, h_ref):
    p = p_ref[...]
    h_ref[...] = jnp.maximum(p[0] + p[1] + b1_ref[...], 0.0)


def _tc3_body(p_ref, w2_ref, b2_ref, out_ref):
    p = p_ref[...]
    q = (p[0] + p[1])[:N]
    z = jnp.dot(q, w2_ref[...], preferred_element_type=jnp.float32)
    z = z + b2_ref[...]
    m = jnp.max(z, axis=1, keepdims=True)
    lse = jnp.log(jnp.sum(jnp.exp(z - m), axis=1, keepdims=True)) + m
    out_ref[...] = z - lse


_tc1 = pl.pallas_call(
    _tc1_body,
    out_shape=jax.ShapeDtypeStruct((N, HID), jnp.float32),
)
_tc2 = pl.pallas_call(
    _tc2_body,
    out_shape=jax.ShapeDtypeStruct((NPAD // 8, 8 * HID), jnp.float32),
)
_tc3 = pl.pallas_call(
    _tc3_body,
    out_shape=jax.ShapeDtypeStruct((N, C), jnp.float32),
)


def kernel(x, edge_index, edge_attr, W1, b1, W2, b2):
    ei = edge_index.reshape(2, NW, NCH, CHUNK)
    ew = edge_attr.reshape(NW, NCH, CHUNK)
    xw = _tc1(x, W1)
    p1, dis, pw = _sc_layer1(ei, ew, xw)
    h = _tc2(p1.reshape(NC, NPAD // 8, 8 * HID),
             jnp.tile(b1, 8)[None, :]).reshape(NPAD, HID)
    p2 = _sc_layer2(ei, pw, h, dis)
    return _tc3(p2, W2, b2[None, :])


# speedup vs baseline: 70.5470x; 1.0467x over previous
import functools

import jax
import jax.numpy as jnp
from jax import lax
from jax.experimental import pallas as pl
from jax.experimental.pallas import tpu as pltpu
from jax.experimental.pallas import tpu_sc as plsc

N = 10000
E = 320000
F_IN = 128
HID = 16
C = 40
CP = 48
NPAD = 10240

NC = 2
NS = 16
NW = NC * NS
CHUNK = 80
NCH = 125
RING = 5
SPAN = NPAD // NS
TSPAN = N // NS


def _mesh():
    return plsc.VectorSubcoreMesh(core_axis_name="c", subcore_axis_name="s")


def _bcast_lane(v16, t):
    idx = jnp.full((16, 1), t, jnp.int32)
    dnums = lax.GatherDimensionNumbers(
        offset_dims=(), collapsed_slice_dims=(0,), start_index_map=(0,))
    return lax.gather(v16, idx, dimension_numbers=dnums, slice_sizes=(1,),
                      mode=lax.GatherScatterMode.PROMISE_IN_BOUNDS)


def _rsqrt16(v):
    i = lax.bitcast_convert_type(v, jnp.int32)
    y = lax.bitcast_convert_type(jnp.int32(0x5F3759DF) - (i >> 1), jnp.float32)
    half_v = v * 0.5
    for _ in range(3):
        y = y * (1.5 - half_v * y * y)
    return jnp.where(v > 0.0, y, 0.0)


@functools.partial(
    pl.kernel,
    out_type=(jax.ShapeDtypeStruct((NC, NPAD, HID), jnp.float32),
              jax.ShapeDtypeStruct((NC, NPAD), jnp.float32),
              jax.ShapeDtypeStruct((NW, NCH, CHUNK), jnp.float32)),
    mesh=_mesh(),
    scratch_types=[
        pltpu.VMEM((NCH, CHUNK), jnp.int32),
        pltpu.VMEM((NCH, CHUNK), jnp.int32),
        pltpu.VMEM((NCH, CHUNK), jnp.float32),
        pltpu.VMEM((NCH, CHUNK), jnp.int32),
        pltpu.VMEM((NCH, CHUNK), jnp.float32),
        pltpu.VMEM((NCH, CHUNK), jnp.float32),
        pltpu.VMEM((RING, CHUNK, HID), jnp.float32),
        pltpu.VMEM((2, CHUNK, HID), jnp.float32),
        pltpu.VMEM((SPAN,), jnp.float32),
        pltpu.VMEM((SPAN, HID), jnp.float32),
        pltpu.VMEM((64, HID), jnp.float32),
        pltpu.VMEM_SHARED((NPAD,), jnp.float32),
        pltpu.VMEM_SHARED((NPAD,), jnp.float32),
        pltpu.VMEM_SHARED((N, HID), jnp.float32),
        pltpu.VMEM_SHARED((NPAD, HID), jnp.float32),
    ] + [pltpu.SemaphoreType.DMA] * (RING + 2 + 3),
    compiler_params=pltpu.CompilerParams(use_tc_tiling_on_sc=False,
                                         disable_bounds_checks=True),
)
def _sc_layer1(ei_hbm, ew_hbm, xw_hbm, p1_hbm, dis_hbm, pw_hbm,
               row_v, col_v, ew_v, col2_v, ew2_v, pw_v, g_v, s_v, db_v, ob_v,
               zb_v, deg_s, dis_s, tab_s, agg_s, *sems):
    gsem = sems[:RING]
    ssem = sems[RING:RING + 2]
    tsem, dsem, psem = sems[RING + 2:]
    c = lax.axis_index("c")
    s = lax.axis_index("s")
    wid = s * NC + c
    wid2 = s * NC + (1 - c)
    r0 = s * SPAN
    t0 = s * TSPAN

    tcopy = pltpu.async_copy(xw_hbm.at[pl.ds(t0, TSPAN), :],
                             tab_s.at[pl.ds(t0, TSPAN), :], tsem)

    def zero_db(i, _):
        db_v[pl.ds(i * 16, 16)] = jnp.zeros((16,), jnp.float32)
        return 0

    lax.fori_loop(0, SPAN // 16, zero_db, 0)
    pltpu.sync_copy(db_v, deg_s.at[pl.ds(r0, SPAN)])

    def zero_zb(i, _):
        zb_v[i, :] = jnp.zeros((16,), jnp.float32)
        return 0

    lax.fori_loop(0, 64, zero_zb, 0)

    def zero_agg(i, _):
        pltpu.sync_copy(zb_v, agg_s.at[pl.ds(r0 + i * 64, 64), :])
        return 0

    lax.fori_loop(0, SPAN // 64, zero_agg, 0)

    pltpu.sync_copy(ei_hbm.at[0, wid], row_v)
    pltpu.sync_copy(ei_hbm.at[1, wid], col_v)
    pltpu.sync_copy(ew_hbm.at[wid], ew_v)
    pltpu.sync_copy(ei_hbm.at[1, wid2], col2_v)
    pltpu.sync_copy(ew_hbm.at[wid2], ew2_v)
    plsc.subcore_barrier()

    def fire_deg(j, _):
        pltpu.async_copy(ew_v.at[j], deg_s.at[col_v.at[j]], dsem, add=True)
        pltpu.async_copy(ew2_v.at[j], deg_s.at[col2_v.at[j]], dsem, add=True)
        return 0

    lax.fori_loop(0, NCH, fire_deg, 0)
    pltpu.make_async_copy(ew_hbm.at[wid], ew_v, dsem).wait()
    pltpu.make_async_copy(ew_hbm.at[wid], ew_v, dsem).wait()
    plsc.subcore_barrier()

    pltpu.sync_copy(deg_s.at[pl.ds(r0, SPAN)], db_v)

    def dis_body(i, _):
        sl = pl.ds(i * 16, 16)
        db_v[sl] = _rsqrt16(db_v[sl])
        return 0

    lax.fori_loop(0, SPAN // 16, dis_body, 0)
    pltpu.sync_copy(db_v, dis_s.at[pl.ds(r0, SPAN)])
    pltpu.sync_copy(db_v, dis_hbm.at[c, pl.ds(r0, SPAN)])
    plsc.subcore_barrier()

    def fire_pw(j, _):
        pltpu.async_copy(dis_s.at[row_v.at[j]], pw_v.at[j], psem)
        return 0

    lax.fori_loop(0, NCH, fire_pw, 0)
    pltpu.make_async_copy(ew_hbm.at[wid], pw_v, psem).wait()

    def pw_mul(j, _):
        for g in range(CHUNK // 16):
            sl = pl.ds(g * 16, 16)
            pw_v[j, sl] = pw_v[j, sl] * ew_v[j, sl]
        return 0

    lax.fori_loop(0, NCH, pw_mul, 0)
    pltpu.sync_copy(pw_v, pw_hbm.at[wid])
    tcopy.wait()
    plsc.subcore_barrier()

    def fire_gather(j, b):
        pltpu.async_copy(tab_s.at[row_v.at[j]], g_v.at[b], gsem[b])

    def fire_scatter(j, sb):
        pltpu.async_copy(s_v.at[sb], agg_s.at[col_v.at[j]], ssem[sb],
                         add=True)

    def wait_g(b):
        pltpu.make_async_copy(xw_hbm.at[pl.ds(0, CHUNK), :], g_v.at[b],
                              gsem[b]).wait()

    def wait_s(sb):
        pltpu.make_async_copy(xw_hbm.at[pl.ds(0, CHUNK), :], s_v.at[sb],
                              ssem[sb]).wait()

    def scale(j, b, sb):
        for g in range(CHUNK // 16):
            w16 = pw_v[j, pl.ds(g * 16, 16)]
            for t in range(16):
                k = g * 16 + t
                wb = _bcast_lane(w16, t)
                s_v[sb, k, :] = g_v[b, k, :] * wb

    for b in range(RING):
        fire_gather(b, b)

    n_groups = NCH // RING

    def body(i, _):
        for b in range(RING):
            j = i * RING + b
            sb = b % 2
            wait_g(b)

            if b < 2:
                @pl.when(i >= 1)
                def _():
                    wait_s(sb)
            else:
                wait_s(sb)

            scale(j, b, sb)
            fire_scatter(j, sb)

            @pl.when(i < n_groups - 1)
            def _():
                fire_gather(j + RING, b)

        return 0

    lax.fori_loop(0, n_groups, body, 0)
    for sb in range(2):
        wait_s(sb)
    plsc.subcore_barrier()
    pltpu.sync_copy(agg_s.at[pl.ds(r0, SPAN), :], ob_v)

    def oscale(g, _):
        dv16 = db_v[pl.ds(g * 16, 16)]
        for t in range(16):
            n = g * 16 + t
            wb = _bcast_lane(dv16, t)
            ob_v[n, :] = ob_v[n, :] * wb
        return 0

    lax.fori_loop(0, SPAN // 16, oscale, 0)
    pltpu.sync_copy(ob_v, p1_hbm.at[c, pl.ds(r0, SPAN), :])


@functools.partial(
    pl.kernel,
    out_type=jax.ShapeDtypeStruct((NC, NPAD, HID), jnp.float32),
    mesh=_mesh(),
    scratch_types=[
        pltpu.VMEM((NCH, CHUNK), jnp.int32),
        pltpu.VMEM((NCH, CHUNK), jnp.int32),
        pltpu.VMEM((NCH, CHUNK), jnp.float32),
        pltpu.VMEM((RING, CHUNK, HID), jnp.float32),
        pltpu.VMEM((2, CHUNK, HID), jnp.float32),
        pltpu.VMEM((SPAN,), jnp.float32),
        pltpu.VMEM((SPAN, HID), jnp.float32),
        pltpu.VMEM((64, HID), jnp.float32),
        pltpu.VMEM_SHARED((NPAD, HID), jnp.float32),
        pltpu.VMEM_SHARED((NPAD, HID), jnp.float32),
    ] + [pltpu.SemaphoreType.DMA] * (RING + 2 + 1),
    compiler_params=pltpu.CompilerParams(use_tc_tiling_on_sc=False,
                                         disable_bounds_checks=True),
)
def _sc_layer2(ei_hbm, ew_hbm, tab_hbm, dis_hbm, out_hbm,
               row_v, col_v, ew_v, g_v, s_v, db_v, ob_v, zb_v, agg_s, tab_s,
               *sems):
    gsem = sems[:RING]
    ssem = sems[RING:RING + 2]
    tsem = sems[RING + 2]
    c = lax.axis_index("c")
    s = lax.axis_index("s")
    wid = s * NC + c
    r0 = s * SPAN

    tcopy = pltpu.async_copy(tab_hbm.at[pl.ds(r0, SPAN), :],
                             tab_s.at[pl.ds(r0, SPAN), :], tsem)

    def zero_zb(i, _):
        zb_v[i, :] = jnp.zeros((16,), jnp.float32)
        return 0

    lax.fori_loop(0, 64, zero_zb, 0)

    def zero_agg(i, _):
        pltpu.sync_copy(zb_v, agg_s.at[pl.ds(r0 + i * 64, 64), :])
        return 0

    lax.fori_loop(0, SPAN // 64, zero_agg, 0)

    pltpu.sync_copy(ei_hbm.at[0, wid], row_v)
    pltpu.sync_copy(ei_hbm.at[1, wid], col_v)
    pltpu.sync_copy(ew_hbm.at[wid], ew_v)
    pltpu.sync_copy(dis_hbm.at[c, pl.ds(r0, SPAN)], db_v)
    tcopy.wait()
    plsc.subcore_barrier()

    def fire_gather(j, b):
        pltpu.async_copy(tab_s.at[row_v.at[j]], g_v.at[b], gsem[b])

    def fire_scatter(j, sb):
        pltpu.async_copy(s_v.at[sb], agg_s.at[col_v.at[j]], ssem[sb],
                         add=True)

    def wait_g(b):
        pltpu.make_async_copy(tab_hbm.at[pl.ds(0, CHUNK), :], g_v.at[b],
                              gsem[b]).wait()

    def wait_s(sb):
        pltpu.make_async_copy(tab_hbm.at[pl.ds(0, CHUNK), :], s_v.at[sb],
                              ssem[sb]).wait()

    def scale(j, b, sb):
        for g in range(CHUNK // 16):
            w16 = ew_v[j, pl.ds(g * 16, 16)]
            for t in range(16):
                k = g * 16 + t
                wb = _bcast_lane(w16, t)
                s_v[sb, k, :] = g_v[b, k, :] * wb

    for b in range(RING):
        fire_gather(b, b)

    n_groups = NCH // RING

    def body(i, _):
        for b in range(RING):
            j = i * RING + b
            sb = b % 2
            wait_g(b)

            if b < 2:
                @pl.when(i >= 1)
                def _():
                    wait_s(sb)
            else:
                wait_s(sb)

            scale(j, b, sb)
            fire_scatter(j, sb)

            @pl.when(i < n_groups - 1)
            def _():
                fire_gather(j + RING, b)

        return 0

    lax.fori_loop(0, n_groups, body, 0)
    for sb in range(2):
        wait_s(sb)
    plsc.subcore_barrier()
    pltpu.sync_copy(agg_s.at[pl.ds(r0, SPAN), :], ob_v)

    def oscale(g, _):
        dv16 = db_v[pl.ds(g * 16, 16)]
        for t in range(16):
            n = g * 16 + t
            wb = _bcast_lane(dv16, t)
            ob_v[n, :] = ob_v[n, :] * wb
        return 0

    lax.fori_loop(0, SPAN // 16, oscale, 0)
    pltpu.sync_copy(ob_v, out_hbm.at[c, pl.ds(r0, SPAN), :])


def _tc1_body(x_ref, w1_ref, xw_ref):
    xw_ref[...] = jnp.dot(x_ref[...], w1_ref[...],
                          preferred_element_type=jnp.float32)


def _tc2_body(p_ref, b1_ref, h_ref):
    p = p_ref[...]
    h_ref[...] = jnp.maximum(p[0] + p[1] + b1_ref[...], 0.0)


def _tc3_body(p_ref, w2bd_ref, b2_ref, out_ref):
    p = p_ref[...]
    q = p[0] + p[1]
    z = jnp.dot(q, w2bd_ref[...], preferred_element_type=jnp.float32)
    z = z + b2_ref[...]
    for k in range(8):
        zk = z[:, k * C:(k + 1) * C]
        m = jnp.max(zk, axis=1, keepdims=True)
        lse = jnp.log(jnp.sum(jnp.exp(zk - m), axis=1, keepdims=True)) + m
        out_ref[:, k * C:(k + 1) * C] = zk - lse


_tc1 = pl.pallas_call(
    _tc1_body,
    out_shape=jax.ShapeDtypeStruct((N, HID), jnp.float32),
)
_tc2 = pl.pallas_call(
    _tc2_body,
    out_shape=jax.ShapeDtypeStruct((NPAD // 8, 8 * HID), jnp.float32),
)
_tc3 = pl.pallas_call(
    _tc3_body,
    out_shape=jax.ShapeDtypeStruct((NPAD // 8, 8 * C), jnp.float32),
)


def kernel(x, edge_index, edge_attr, W1, b1, W2, b2):
    ei = edge_index.reshape(2, NW, NCH, CHUNK)
    ew = edge_attr.reshape(NW, NCH, CHUNK)
    xw = _tc1(x, W1)
    p1, dis, pw = _sc_layer1(ei, ew, xw)
    h = _tc2(p1.reshape(NC, NPAD // 8, 8 * HID),
             jnp.tile(b1, 8)[None, :]).reshape(NPAD, HID)
    p2 = _sc_layer2(ei, pw, h, dis)
    w2bd = jax.scipy.linalg.block_diag(*([W2] * 8))
    b2t = jnp.tile(b2, 8)[None, :]
    out8 = _tc3(p2.reshape(NC, NPAD // 8, 8 * HID), w2bd, b2t)
    return out8.reshape(NPAD, C)[:N]


# speedup vs baseline: 70.6547x; 1.0015x over previous
import functools

import jax
import jax.numpy as jnp
from jax import lax
from jax.experimental import pallas as pl
from jax.experimental.pallas import tpu as pltpu
from jax.experimental.pallas import tpu_sc as plsc

N = 10000
E = 320000
F_IN = 128
HID = 16
C = 40
CP = 48
NPAD = 10240

NC = 2
NS = 16
NW = NC * NS
CHUNK = 80
NCH = 125
RING = 5
SPAN = NPAD // NS
TSPAN = N // NS


def _mesh():
    return plsc.VectorSubcoreMesh(core_axis_name="c", subcore_axis_name="s")


def _bcast_lane(v16, t):
    idx = jnp.full((16, 1), t, jnp.int32)
    dnums = lax.GatherDimensionNumbers(
        offset_dims=(), collapsed_slice_dims=(0,), start_index_map=(0,))
    return lax.gather(v16, idx, dimension_numbers=dnums, slice_sizes=(1,),
                      mode=lax.GatherScatterMode.PROMISE_IN_BOUNDS)


def _rsqrt16(v):
    i = lax.bitcast_convert_type(v, jnp.int32)
    y = lax.bitcast_convert_type(jnp.int32(0x5F3759DF) - (i >> 1), jnp.float32)
    half_v = v * 0.5
    for _ in range(3):
        y = y * (1.5 - half_v * y * y)
    return jnp.where(v > 0.0, y, 0.0)


@functools.partial(
    pl.kernel,
    out_type=(jax.ShapeDtypeStruct((NC, NPAD), jnp.float32),
              jax.ShapeDtypeStruct((NW, NCH, CHUNK), jnp.float32)),
    mesh=_mesh(),
    scratch_types=[
        pltpu.VMEM((NCH, CHUNK), jnp.int32),
        pltpu.VMEM((NCH, CHUNK), jnp.int32),
        pltpu.VMEM((NCH, CHUNK), jnp.float32),
        pltpu.VMEM((NCH, CHUNK), jnp.int32),
        pltpu.VMEM((NCH, CHUNK), jnp.float32),
        pltpu.VMEM((NCH, CHUNK), jnp.float32),
        pltpu.VMEM((SPAN,), jnp.float32),
        pltpu.VMEM_SHARED((NPAD,), jnp.float32),
        pltpu.VMEM_SHARED((NPAD,), jnp.float32),
    ] + [pltpu.SemaphoreType.DMA] * 2,
    compiler_params=pltpu.CompilerParams(use_tc_tiling_on_sc=False,
                                         disable_bounds_checks=True),
)
def _sc_prep(ei_hbm, ew_hbm, dis_hbm, pw_hbm,
             row_v, col_v, ew_v, col2_v, ew2_v, pw_v, db_v,
             deg_s, dis_s, *sems):
    dsem, psem = sems
    c = lax.axis_index("c")
    s = lax.axis_index("s")
    wid = s * NC + c
    wid2 = s * NC + (1 - c)
    r0 = s * SPAN

    def zero_db(i, _):
        db_v[pl.ds(i * 16, 16)] = jnp.zeros((16,), jnp.float32)
        return 0

    lax.fori_loop(0, SPAN // 16, zero_db, 0)
    pltpu.sync_copy(db_v, deg_s.at[pl.ds(r0, SPAN)])

    pltpu.sync_copy(ei_hbm.at[0, wid], row_v)
    pltpu.sync_copy(ei_hbm.at[1, wid], col_v)
    pltpu.sync_copy(ew_hbm.at[wid], ew_v)
    pltpu.sync_copy(ei_hbm.at[1, wid2], col2_v)
    pltpu.sync_copy(ew_hbm.at[wid2], ew2_v)
    plsc.subcore_barrier()

    def fire_deg(j, _):
        pltpu.async_copy(ew_v.at[j], deg_s.at[col_v.at[j]], dsem, add=True)
        pltpu.async_copy(ew2_v.at[j], deg_s.at[col2_v.at[j]], dsem, add=True)
        return 0

    lax.fori_loop(0, NCH, fire_deg, 0)
    pltpu.make_async_copy(ew_hbm.at[wid], ew_v, dsem).wait()
    pltpu.make_async_copy(ew_hbm.at[wid], ew_v, dsem).wait()
    plsc.subcore_barrier()

    pltpu.sync_copy(deg_s.at[pl.ds(r0, SPAN)], db_v)

    def dis_body(i, _):
        sl = pl.ds(i * 16, 16)
        db_v[sl] = _rsqrt16(db_v[sl])
        return 0

    lax.fori_loop(0, SPAN // 16, dis_body, 0)
    pltpu.sync_copy(db_v, dis_s.at[pl.ds(r0, SPAN)])
    pltpu.sync_copy(db_v, dis_hbm.at[c, pl.ds(r0, SPAN)])
    plsc.subcore_barrier()

    def fire_pw(j, _):
        pltpu.async_copy(dis_s.at[row_v.at[j]], pw_v.at[j], psem)
        return 0

    lax.fori_loop(0, NCH, fire_pw, 0)
    pltpu.make_async_copy(ew_hbm.at[wid], pw_v, psem).wait()

    def pw_mul(j, _):
        for g in range(CHUNK // 16):
            sl = pl.ds(g * 16, 16)
            pw_v[j, sl] = pw_v[j, sl] * ew_v[j, sl]
        return 0

    lax.fori_loop(0, NCH, pw_mul, 0)
    pltpu.sync_copy(pw_v, pw_hbm.at[wid])


@functools.partial(
    pl.kernel,
    out_type=jax.ShapeDtypeStruct((NC, NPAD, HID), jnp.float32),
    mesh=_mesh(),
    scratch_types=[
        pltpu.VMEM((NCH, CHUNK), jnp.int32),
        pltpu.VMEM((NCH, CHUNK), jnp.int32),
        pltpu.VMEM((NCH, CHUNK), jnp.float32),
        pltpu.VMEM((RING, CHUNK, HID), jnp.float32),
        pltpu.VMEM((2, CHUNK, HID), jnp.float32),
        pltpu.VMEM((SPAN,), jnp.float32),
        pltpu.VMEM((SPAN, HID), jnp.float32),
        pltpu.VMEM((64, HID), jnp.float32),
        pltpu.VMEM_SHARED((NPAD, HID), jnp.float32),
        pltpu.VMEM_SHARED((NPAD, HID), jnp.float32),
    ] + [pltpu.SemaphoreType.DMA] * (RING + 2 + 1),
    compiler_params=pltpu.CompilerParams(use_tc_tiling_on_sc=False,
                                         disable_bounds_checks=True),
)
def _sc_prop(ei_hbm, ew_hbm, tab_hbm, dis_hbm, out_hbm,
               row_v, col_v, ew_v, g_v, s_v, db_v, ob_v, zb_v, agg_s, tab_s,
               *sems):
    gsem = sems[:RING]
    ssem = sems[RING:RING + 2]
    tsem = sems[RING + 2]
    c = lax.axis_index("c")
    s = lax.axis_index("s")
    wid = s * NC + c
    r0 = s * SPAN

    tcopy = pltpu.async_copy(tab_hbm.at[pl.ds(r0, SPAN), :],
                             tab_s.at[pl.ds(r0, SPAN), :], tsem)

    def zero_zb(i, _):
        zb_v[i, :] = jnp.zeros((16,), jnp.float32)
        return 0

    lax.fori_loop(0, 64, zero_zb, 0)

    def zero_agg(i, _):
        pltpu.sync_copy(zb_v, agg_s.at[pl.ds(r0 + i * 64, 64), :])
        return 0

    lax.fori_loop(0, SPAN // 64, zero_agg, 0)

    pltpu.sync_copy(ei_hbm.at[0, wid], row_v)
    pltpu.sync_copy(ei_hbm.at[1, wid], col_v)
    pltpu.sync_copy(ew_hbm.at[wid], ew_v)
    pltpu.sync_copy(dis_hbm.at[c, pl.ds(r0, SPAN)], db_v)
    tcopy.wait()
    plsc.subcore_barrier()

    def fire_gather(j, b):
        pltpu.async_copy(tab_s.at[row_v.at[j]], g_v.at[b], gsem[b])

    def fire_scatter(j, sb):
        pltpu.async_copy(s_v.at[sb], agg_s.at[col_v.at[j]], ssem[sb],
                         add=True)

    def wait_g(b):
        pltpu.make_async_copy(tab_hbm.at[pl.ds(0, CHUNK), :], g_v.at[b],
                              gsem[b]).wait()

    def wait_s(sb):
        pltpu.make_async_copy(tab_hbm.at[pl.ds(0, CHUNK), :], s_v.at[sb],
                              ssem[sb]).wait()

    def scale(j, b, sb):
        for g in range(CHUNK // 16):
            w16 = ew_v[j, pl.ds(g * 16, 16)]
            for t in range(16):
                k = g * 16 + t
                wb = _bcast_lane(w16, t)
                s_v[sb, k, :] = g_v[b, k, :] * wb

    for b in range(RING):
        fire_gather(b, b)

    n_groups = NCH // RING

    def body(i, _):
        for b in range(RING):
            j = i * RING + b
            sb = b % 2
            wait_g(b)

            if b < 2:
                @pl.when(i >= 1)
                def _():
                    wait_s(sb)
            else:
                wait_s(sb)

            scale(j, b, sb)
            fire_scatter(j, sb)

            @pl.when(i < n_groups - 1)
            def _():
                fire_gather(j + RING, b)

        return 0

    lax.fori_loop(0, n_groups, body, 0)
    for sb in range(2):
        wait_s(sb)
    plsc.subcore_barrier()
    pltpu.sync_copy(agg_s.at[pl.ds(r0, SPAN), :], ob_v)

    def oscale(g, _):
        dv16 = db_v[pl.ds(g * 16, 16)]
        for t in range(16):
            n = g * 16 + t
            wb = _bcast_lane(dv16, t)
            ob_v[n, :] = ob_v[n, :] * wb
        return 0

    lax.fori_loop(0, SPAN // 16, oscale, 0)
    pltpu.sync_copy(ob_v, out_hbm.at[c, pl.ds(r0, SPAN), :])


def _tc1_body(x_ref, w1_ref, xw_ref):
    xw_ref[:N, :] = jnp.dot(x_ref[...], w1_ref[...],
                            preferred_element_type=jnp.float32)


def _tc2_body(p_ref, b1_ref, h_ref):
    p = p_ref[...]
    h_ref[...] = jnp.maximum(p[0] + p[1] + b1_ref[...], 0.0)


def _tc3_body(p_ref, w2bd_ref, b2_ref, out_ref):
    p = p_ref[...]
    q = p[0] + p[1]
    z = jnp.dot(q, w2bd_ref[...], preferred_element_type=jnp.float32)
    z = z + b2_ref[...]
    for k in range(8):
        zk = z[:, k * C:(k + 1) * C]
        m = jnp.max(zk, axis=1, keepdims=True)
        lse = jnp.log(jnp.sum(jnp.exp(zk - m), axis=1, keepdims=True)) + m
        out_ref[:, k * C:(k + 1) * C] = zk - lse


_tc1 = pl.pallas_call(
    _tc1_body,
    out_shape=jax.ShapeDtypeStruct((NPAD, HID), jnp.float32),
)
_tc2 = pl.pallas_call(
    _tc2_body,
    out_shape=jax.ShapeDtypeStruct((NPAD // 8, 8 * HID), jnp.float32),
)
_tc3 = pl.pallas_call(
    _tc3_body,
    out_shape=jax.ShapeDtypeStruct((NPAD // 8, 8 * C), jnp.float32),
)


def kernel(x, edge_index, edge_attr, W1, b1, W2, b2):
    ei = edge_index.reshape(2, NW, NCH, CHUNK)
    ew = edge_attr.reshape(NW, NCH, CHUNK)
    dis, pw = _sc_prep(ei, ew)
    xw = _tc1(x, W1)
    p1 = _sc_prop(ei, pw, xw, dis)
    h = _tc2(p1.reshape(NC, NPAD // 8, 8 * HID),
             jnp.tile(b1, 8)[None, :]).reshape(NPAD, HID)
    p2 = _sc_prop(ei, pw, h, dis)
    w2bd = jax.scipy.linalg.block_diag(*([W2] * 8))
    b2t = jnp.tile(b2, 8)[None, :]
    out8 = _tc3(p2.reshape(NC, NPAD // 8, 8 * HID), w2bd, b2t)
    return out8.reshape(NPAD, C)[:N]
